# baseline (speedup 1.0000x reference)
import functools
import math

import jax
import jax.numpy as jnp
from jax import lax
from jax.experimental import pallas as pl
from jax.experimental.pallas import tpu as pltpu
from jax.experimental.pallas import tpu_sc as plsc

_SCALE = 8.0
_LANES = 16
_NBUF = 3


@functools.cache
def _build(B, V, D):
    NC, NS = 2, 16
    NW = NC * NS
    assert B % NW == 0
    b_per_w = B // NW
    C = 128
    assert b_per_w % C == 0
    n_chunks = b_per_w // C
    OB = (C * D) // 128
    main_end = 1 + ((n_chunks - 3 - 1) // _NBUF) * _NBUF
    assert main_end >= 1 and main_end + 2 <= n_chunks

    mesh = plsc.VectorSubcoreMesh(core_axis_name="c", subcore_axis_name="s")

    @functools.partial(
        pl.kernel,
        mesh=mesh,
        compiler_params=pltpu.CompilerParams(needs_layout_passes=False),
        out_type=jax.ShapeDtypeStruct(((B * D) // 128, 128), jnp.float32),
        scratch_types=[
            [pltpu.VMEM((C,), jnp.int32) for _ in range(_NBUF)],
            [pltpu.VMEM((C,), jnp.int32) for _ in range(_NBUF)],
            [pltpu.VMEM((C, 128), jnp.float32) for _ in range(_NBUF)],
            [pltpu.VMEM((OB, 128), jnp.float32) for _ in range(_NBUF)],
            [pltpu.SemaphoreType.DMA for _ in range(_NBUF)],
            [pltpu.SemaphoreType.DMA for _ in range(_NBUF)],
        ],
    )
    def emb(x_hbm, lut_hbm, out_hbm, idx_v, u_v, rows_v, obuf, gsem, ssem):
        wid = lax.axis_index("s") * NC + lax.axis_index("c")
        base = wid * b_per_w
        iota = lax.iota(jnp.int32, _LANES)

        def load_and_gather(c, b):
            off = base + c * C
            pltpu.sync_copy(x_hbm.at[pl.ds(off, C)], idx_v[b])

            @plsc.parallel_loop(0, C // _LANES, unroll=4)
            def _(g):
                sl = pl.ds(g * _LANES, _LANES)
                u_v[b][sl] = lax.shift_right_logical(idx_v[b][sl], 1)

            pltpu.async_copy(lut_hbm.at[u_v[b]], rows_v[b], gsem[b])

        def drain_gather(b):
            pltpu.make_async_copy(
                lut_hbm.at[pl.ds(0, C)], rows_v[b], gsem[b]
            ).wait()

        def start_store(c, b):
            off2 = pl.multiple_of((base + c * C) * D // 128, 8)
            pltpu.async_copy(obuf[b], out_hbm.at[pl.ds(off2, OB)], ssem[b])

        def drain_store(b):
            pltpu.make_async_copy(
                obuf[b], out_hbm.at[pl.ds(0, OB)], ssem[b]
            ).wait()

        def scale(b):
            @plsc.parallel_loop(0, C, unroll=2)
            def _(r):
                rsplat = jnp.full((_LANES,), 0, jnp.int32) + r
                pvec = lax.bitwise_and(
                    plsc.load_gather(idx_v[b], [rsplat]), 1
                )
                cbase = pvec * D
                for k in range(D // _LANES):
                    col = cbase + (k * _LANES) + iota
                    val = plsc.load_gather(rows_v[b], [rsplat, col])
                    i = r * (D // _LANES) + k
                    sl = pl.ds((i % 8) * _LANES, _LANES)
                    obuf[b][i // 8, sl] = val * _SCALE

        load_and_gather(0, 0)
        load_and_gather(1, 1)
        drain_gather(0)
        load_and_gather(2, 2)
        scale(0)
        start_store(0, 0)

        @pl.loop(1, main_end, step=_NBUF)
        def _(i):
            for b_off in range(_NBUF):
                c = i + b_off
                b = (1 + b_off) % _NBUF
                nb = (b + 2) % _NBUF
                drain_gather(b)
                drain_store(nb)
                load_and_gather(c + 2, nb)
                scale(b)
                start_store(c, b)

        for c in range(main_end, n_chunks):
            b = c % _NBUF
            nb = (b + 2) % _NBUF
            drain_gather(b)
            if c + 2 < n_chunks:
                drain_store(nb)
                load_and_gather(c + 2, nb)
            scale(b)
            start_store(c, b)
        for c in range(n_chunks - _NBUF, n_chunks):
            drain_store(c % _NBUF)

    return emb


def kernel(x, lut):
    B0, S = x.shape
    V, D = lut.shape
    B = B0 * S
    xf = x.reshape(B).astype(jnp.int32)
    lutr = lut.reshape(V // 2, 2 * D)
    out = _build(B, V, D)(xf, lutr)
    return out.reshape(B0, S, D)


# speedup vs baseline: 1.1004x; 1.1004x over previous
import functools
import math

import jax
import jax.numpy as jnp
from jax import lax
from jax.experimental import pallas as pl
from jax.experimental.pallas import tpu as pltpu
from jax.experimental.pallas import tpu_sc as plsc

_SCALE = 8.0
_LANES = 16
_NBUF = 3


@functools.cache
def _build(B, V, D):
    NC, NS = 2, 16
    NW = NC * NS
    assert B % NW == 0
    b_per_w = B // NW
    C = 512
    assert b_per_w % C == 0 and C % 128 == 0
    n_chunks = b_per_w // C
    K = C // 128
    vecs_per_row = D // _LANES
    main_end = 1 + ((n_chunks - 3 - 1) // _NBUF) * _NBUF
    assert main_end >= 1 and (main_end - 1) % _NBUF == 0 and main_end + 2 <= n_chunks

    mesh = plsc.VectorSubcoreMesh(core_axis_name="c", subcore_axis_name="s")

    @functools.partial(
        pl.kernel,
        mesh=mesh,
        compiler_params=pltpu.CompilerParams(use_tc_tiling_on_sc=False),
        out_type=jax.ShapeDtypeStruct((B, D), jnp.float32),
        scratch_types=[
            [pltpu.VMEM((C,), jnp.int32) for _ in range(_NBUF)],
            [pltpu.VMEM((C, D), jnp.float32) for _ in range(_NBUF)],
            [pltpu.SemaphoreType.DMA for _ in range(_NBUF)],
            [pltpu.SemaphoreType.DMA for _ in range(_NBUF)],
        ],
    )
    def emb(x_hbm, lut_hbm, out_hbm, idx_v, rows_v, gsem, ssem):
        wid = lax.axis_index("s") * NC + lax.axis_index("c")
        base = wid * b_per_w

        def load_and_gather(c, b):
            pltpu.sync_copy(x_hbm.at[pl.ds(base + c * C, C)], idx_v[b])
            for j in range(K):
                sl = pl.ds(j * 128, 128)
                pltpu.async_copy(
                    lut_hbm.at[idx_v[b].at[sl]], rows_v[b].at[sl], gsem[b]
                )

        def drain_gather(b):
            pltpu.make_async_copy(
                lut_hbm.at[pl.ds(0, C)], rows_v[b], gsem[b]
            ).wait()

        def start_store(c, b):
            pltpu.async_copy(
                rows_v[b], out_hbm.at[pl.ds(base + c * C, C)], ssem[b]
            )

        def drain_store(b):
            pltpu.make_async_copy(
                rows_v[b], out_hbm.at[pl.ds(base, C)], ssem[b]
            ).wait()

        def scale(b):
            @plsc.parallel_loop(0, C, unroll=8)
            def _(r):
                for k in range(vecs_per_row):
                    sl = pl.ds(k * _LANES, _LANES)
                    rows_v[b][r, sl] = rows_v[b][r, sl] * _SCALE

        load_and_gather(0, 0)
        load_and_gather(1, 1)
        drain_gather(0)
        load_and_gather(2, 2)
        scale(0)
        start_store(0, 0)

        @pl.loop(1, main_end, step=_NBUF)
        def _(i):
            for b_off in range(_NBUF):
                c = i + b_off
                b = (1 + b_off) % _NBUF
                nb = (b + 2) % _NBUF
                drain_gather(b)
                drain_store(nb)
                load_and_gather(c + 2, nb)
                scale(b)
                start_store(c, b)

        for c in range(main_end, n_chunks):
            b = c % _NBUF
            nb = (b + 2) % _NBUF
            drain_gather(b)
            if c + 2 < n_chunks:
                drain_store(nb)
                load_and_gather(c + 2, nb)
            scale(b)
            start_store(c, b)
        for c in range(n_chunks - _NBUF, n_chunks):
            drain_store(c % _NBUF)

    return emb


def kernel(x, lut):
    B0, S = x.shape
    V, D = lut.shape
    B = B0 * S
    xf = x.reshape(B).astype(jnp.int32)
    out = _build(B, V, D)(xf, lut)
    return out.reshape(B0, S, D)
